# baseline (speedup 1.0000x reference)
import functools

import jax
import jax.numpy as jnp
from jax import lax
from jax.experimental import pallas as pl
from jax.experimental.pallas import tpu as pltpu
from jax.experimental.pallas import tpu_sc as plsc

NC = 2
NS = 16
NW = NC * NS
CH = 128
NB = 4

_HI = jax.lax.Precision.HIGHEST
_GDN = lax.GatherDimensionNumbers(
    offset_dims=(), collapsed_slice_dims=(0,), start_index_map=(0,))


def _spmm_sc(z, rows2d, cols2d, vals2d, zeros_hbm, n_pad):
    F = z.shape[1]
    K = rows2d.shape[0] // NW
    npad = n_pad
    rpt = npad // NS
    mesh = plsc.VectorSubcoreMesh(core_axis_name="c", subcore_axis_name="s")

    @functools.partial(
        pl.kernel,
        mesh=mesh,
        compiler_params=pltpu.CompilerParams(use_tc_tiling_on_sc=False),
        out_type=jax.ShapeDtypeStruct((NC, npad, F), jnp.float32),
        scratch_types=[
            pltpu.VMEM((K, CH), jnp.int32),
            pltpu.VMEM((K, CH), jnp.int32),
            pltpu.VMEM((K * CH,), jnp.float32),
            pltpu.VMEM((NB, CH, F), jnp.float32),
            pltpu.VMEM_SHARED((npad, F), jnp.float32),
            pltpu.SemaphoreType.DMA((NB,)),
            pltpu.SemaphoreType.DMA((NB,)),
        ],
    )
    def k(z_hbm, rows_hbm, cols_hbm, vals_hbm, zer_hbm, out_hbm,
          cols_v, rows_v, vals_v, gbuf, acc, sem_g, sem_s):
        c = lax.axis_index("c")
        s = lax.axis_index("s")
        wid = s * NC + c

        pltpu.sync_copy(cols_hbm.at[pl.ds(wid * K, K)], cols_v)
        pltpu.sync_copy(rows_hbm.at[pl.ds(wid * K, K)], rows_v)
        pltpu.sync_copy(vals_hbm.at[pl.ds(wid * K * CH, K * CH)], vals_v)

        pltpu.sync_copy(zer_hbm.at[pl.ds(s * rpt, rpt)],
                        acc.at[pl.ds(s * rpt, rpt)])
        plsc.subcore_barrier()

        def g_issue(j, b):
            pltpu.async_copy(z_hbm.at[cols_v.at[j]], gbuf.at[b], sem_g.at[b])

        def g_wait(j, b):
            pltpu.make_async_copy(
                z_hbm.at[cols_v.at[j]], gbuf.at[b], sem_g.at[b]).wait()

        def s_issue(j, b):
            pltpu.async_copy(gbuf.at[b], acc.at[rows_v.at[j]], sem_s.at[b],
                             add=True)

        def s_wait(j, b):
            pltpu.make_async_copy(
                gbuf.at[b], acc.at[rows_v.at[j]], sem_s.at[b]).wait()

        def scale(j, b):
            def edge16(g, icarry):
                vv = vals_v[pl.ds(j * CH + g * 16, 16)]
                for e in range(16):
                    v = lax.gather(
                        vv, jnp.full((16, 1), e, jnp.int32),
                        _GDN, slice_sizes=(1,),
                        mode=lax.GatherScatterMode.PROMISE_IN_BOUNDS)
                    row = g * 16 + e
                    for cc in range(F // 16):
                        gbuf[b, row, pl.ds(cc * 16, 16)] = (
                            gbuf[b, row, pl.ds(cc * 16, 16)] * v)
                return icarry
            lax.fori_loop(0, CH // 16, edge16, 0)

        H = NB // 2
        for b in range(H):
            g_issue(b, b)
        for j in range(H):
            b = j % NB
            g_wait(j, b)
            scale(j, b)
            s_issue(j, b)
            g_issue(j + H, (j + H) % NB)

        def main(o, carry):
            for bi in range(NB):
                j = H + o * NB + bi
                b = (H + bi) % NB
                g_wait(j, b)
                scale(j, b)
                s_issue(j, b)
                b2 = (b + H) % NB
                s_wait(j - H, b2)
                g_issue(j + H, b2)
            return carry
        lax.fori_loop(0, (K - 2 * H) // NB, main, 0)

        for t in range(H):
            j = K - H + t
            b = j % NB
            g_wait(j, b)
            scale(j, b)
            s_issue(j, b)
            s_wait(j - H, (j - H) % NB)
        for t in range(H):
            j = K - H + t
            s_wait(j, j % NB)
        plsc.subcore_barrier()

        pltpu.sync_copy(acc.at[pl.ds(s * rpt, rpt)],
                        out_hbm.at[c, pl.ds(s * rpt, rpt)])

    return k(z, rows2d, cols2d, vals2d, zeros_hbm)


def _tc_in(features, Ws, bs, W0):
    n, d = features.shape
    h = Ws.shape[1]
    mid = W0.shape[1]
    bn = 2000

    def body(f_ref, ws_ref, bs_ref, w0_ref, o_ref):
        x = jnp.dot(f_ref[...], ws_ref[...],
                    preferred_element_type=jnp.float32, precision=_HI)
        x = x + bs_ref[...]
        o_ref[...] = jnp.dot(x, w0_ref[...],
                             preferred_element_type=jnp.float32, precision=_HI)

    return pl.pallas_call(
        body,
        grid=(n // bn,),
        in_specs=[
            pl.BlockSpec((bn, d), lambda i: (i, 0)),
            pl.BlockSpec((d, h), lambda i: (0, 0)),
            pl.BlockSpec((1, h), lambda i: (0, 0)),
            pl.BlockSpec((h, mid), lambda i: (0, 0)),
        ],
        out_specs=pl.BlockSpec((bn, mid), lambda i: (i, 0)),
        out_shape=jax.ShapeDtypeStruct((n, mid), jnp.float32),
    )(features, Ws, bs, W0)


def _tc_relu(p0, p1, b0, n):
    mid = p0.shape[1]
    bn = 2000

    def body(a_ref, b_ref, bias_ref, o_ref):
        o_ref[...] = jnp.maximum(a_ref[...] + b_ref[...] + bias_ref[...], 0.0)

    return pl.pallas_call(
        body,
        grid=(n // bn,),
        in_specs=[
            pl.BlockSpec((bn, mid), lambda i: (i, 0)),
            pl.BlockSpec((bn, mid), lambda i: (i, 0)),
            pl.BlockSpec((1, mid), lambda i: (0, 0)),
        ],
        out_specs=pl.BlockSpec((bn, mid), lambda i: (i, 0)),
        out_shape=jax.ShapeDtypeStruct((n, mid), jnp.float32),
    )(p0, p1, b0)


def _tc_out(q0, q1, W1, b1, n):
    mid = q0.shape[1]
    h = W1.shape[1]
    bn = 2000

    def body(a_ref, b_ref, w_ref, bias_ref, o_ref):
        x = a_ref[...] + b_ref[...]
        o_ref[...] = jnp.dot(x, w_ref[...],
                             preferred_element_type=jnp.float32,
                             precision=_HI) + bias_ref[...]

    return pl.pallas_call(
        body,
        grid=(n // bn,),
        in_specs=[
            pl.BlockSpec((bn, mid), lambda i: (i, 0)),
            pl.BlockSpec((bn, mid), lambda i: (i, 0)),
            pl.BlockSpec((mid, h), lambda i: (0, 0)),
            pl.BlockSpec((1, h), lambda i: (0, 0)),
        ],
        out_specs=pl.BlockSpec((bn, h), lambda i: (i, 0)),
        out_shape=jax.ShapeDtypeStruct((n, h), jnp.float32),
    )(q0, q1, W1, b1)


def kernel(features, rows, cols, vals, W_start, b_start, W0, b0, W1, b1):
    n = features.shape[0]
    nnz = rows.shape[0]
    k_per_w = -(-nnz // (NW * CH))
    k_per_w = -(-k_per_w // 8) * 8
    nnz_pad = NW * k_per_w * CH
    n_pad = -(-n // (NS * 8)) * (NS * 8)
    pad = nnz_pad - nnz
    rows2d = jnp.pad(rows, (0, pad)).reshape(NW * k_per_w, CH)
    cols2d = jnp.pad(cols, (0, pad)).reshape(NW * k_per_w, CH)
    vals1d = jnp.pad(vals, (0, pad))

    zeros_hbm = jnp.zeros((n_pad, W0.shape[1]), jnp.float32)
    z = _tc_in(features, W_start, b_start.reshape(1, -1), W0)
    p = _spmm_sc(z, rows2d, cols2d, vals1d, zeros_hbm, n_pad)
    h = _tc_relu(p[0], p[1], b0.reshape(1, -1), n)
    q = _spmm_sc(h, rows2d, cols2d, vals1d, zeros_hbm, n_pad)
    return _tc_out(q[0], q[1], W1, b1.reshape(1, -1), n)


# speedup vs baseline: 3.2897x; 3.2897x over previous
import functools

import jax
import jax.numpy as jnp
from jax import lax
from jax.experimental import pallas as pl
from jax.experimental.pallas import tpu as pltpu
from jax.experimental.pallas import tpu_sc as plsc

NC = 2
NS = 16
NW = NC * NS
CH = 128
NB = 4

_HI = jax.lax.Precision.HIGHEST
_GDN = lax.GatherDimensionNumbers(
    offset_dims=(), collapsed_slice_dims=(0,), start_index_map=(0,))


def _spmm_sc(z, rows2d, cols2d, vals2d, zeros_hbm, n_pad):
    F = z.shape[1]
    K = rows2d.shape[0] // NW
    npad = n_pad
    rpt = npad // NS
    mesh = plsc.VectorSubcoreMesh(core_axis_name="c", subcore_axis_name="s")

    @functools.partial(
        pl.kernel,
        mesh=mesh,
        compiler_params=pltpu.CompilerParams(use_tc_tiling_on_sc=False),
        out_type=jax.ShapeDtypeStruct((NC, npad, F), jnp.float32),
        scratch_types=[
            pltpu.VMEM((NB, CH), jnp.int32),
            pltpu.VMEM((NB, CH), jnp.int32),
            pltpu.VMEM((K * CH,), jnp.float32),
            pltpu.VMEM((NB, CH, F), jnp.float32),
            pltpu.VMEM_SHARED((npad, F), jnp.float32),
            pltpu.VMEM_SHARED((npad, F), jnp.float32),
            pltpu.SemaphoreType.DMA((NB,)),
            pltpu.SemaphoreType.DMA((NB,)),
            pltpu.SemaphoreType.DMA((NB,)),
        ],
    )
    def k(z_hbm, rows_hbm, cols_hbm, vals_hbm, zer_hbm, out_hbm,
          colb, rowb, vals_v, gbuf, zs, acc, sem_i, sem_g, sem_s):
        c = lax.axis_index("c")
        s = lax.axis_index("s")
        wid = s * NC + c

        pltpu.sync_copy(vals_hbm.at[pl.ds(wid * K * CH, K * CH)], vals_v)
        pltpu.sync_copy(z_hbm.at[pl.ds(s * rpt, rpt)],
                        zs.at[pl.ds(s * rpt, rpt)])
        pltpu.sync_copy(zer_hbm.at[pl.ds(s * rpt, rpt)],
                        acc.at[pl.ds(s * rpt, rpt)])
        plsc.subcore_barrier()

        def i_issue(j, b):
            pltpu.async_copy(cols_hbm.at[wid * K + j], colb.at[b], sem_i.at[b])
            pltpu.async_copy(rows_hbm.at[wid * K + j], rowb.at[b], sem_i.at[b])

        def i_wait(j, b):
            pltpu.make_async_copy(
                cols_hbm.at[wid * K + j], colb.at[b], sem_i.at[b]).wait()
            pltpu.make_async_copy(
                rows_hbm.at[wid * K + j], rowb.at[b], sem_i.at[b]).wait()

        def g_issue(j, b):
            pltpu.async_copy(zs.at[colb.at[b]], gbuf.at[b], sem_g.at[b])

        def g_wait(j, b):
            pltpu.make_async_copy(
                zs.at[colb.at[b]], gbuf.at[b], sem_g.at[b]).wait()

        def s_issue(j, b):
            pltpu.async_copy(gbuf.at[b], acc.at[rowb.at[b]], sem_s.at[b],
                             add=True)

        def s_wait(j, b):
            pltpu.make_async_copy(
                gbuf.at[b], acc.at[rowb.at[b]], sem_s.at[b]).wait()

        def scale(j, b):
            def edge16(g, icarry):
                vv = vals_v[pl.ds(j * CH + g * 16, 16)]
                for e in range(16):
                    v = lax.gather(
                        vv, jnp.full((16, 1), e, jnp.int32),
                        _GDN, slice_sizes=(1,),
                        mode=lax.GatherScatterMode.PROMISE_IN_BOUNDS)
                    row = g * 16 + e
                    for cc in range(F // 16):
                        gbuf[b, row, pl.ds(cc * 16, 16)] = (
                            gbuf[b, row, pl.ds(cc * 16, 16)] * v)
                return icarry
            lax.fori_loop(0, CH // 16, edge16, 0)

        i_issue(0, 0)
        i_issue(1, 1)
        i_wait(0, 0)
        g_issue(0, 0)
        for j in range(2):
            b = j % NB
            g_wait(j, b)
            scale(j, b)
            s_issue(j, b)
            i_issue(j + 2, (j + 2) % NB)
            i_wait(j + 1, (j + 1) % NB)
            g_issue(j + 1, (j + 1) % NB)

        def main(o, carry):
            for bi in range(NB):
                j = 2 + o * NB + bi
                b = (2 + bi) % NB
                g_wait(j, b)
                scale(j, b)
                s_issue(j, b)
                b2 = (b + 2) % NB
                s_wait(j - 2, b2)
                i_issue(j + 2, b2)
                b1 = (b + 1) % NB
                i_wait(j + 1, b1)
                g_issue(j + 1, b1)
            return carry
        lax.fori_loop(0, (K - 4) // NB, main, 0)

        j = K - 2
        b = j % NB
        g_wait(j, b)
        scale(j, b)
        s_issue(j, b)
        s_wait(j - 2, (j - 2) % NB)
        i_wait(j + 1, (j + 1) % NB)
        g_issue(j + 1, (j + 1) % NB)

        j = K - 1
        b = j % NB
        g_wait(j, b)
        scale(j, b)
        s_issue(j, b)
        s_wait(j - 2, (j - 2) % NB)

        s_wait(K - 2, (K - 2) % NB)
        s_wait(K - 1, (K - 1) % NB)
        plsc.subcore_barrier()

        pltpu.sync_copy(acc.at[pl.ds(s * rpt, rpt)],
                        out_hbm.at[c, pl.ds(s * rpt, rpt)])

    return k(z, rows2d, cols2d, vals2d, zeros_hbm)


def _tc_in(features, Ws, bs, W0):
    n, d = features.shape
    h = Ws.shape[1]
    mid = W0.shape[1]
    bn = 2048

    def body(f_ref, ws_ref, bs_ref, w0_ref, o_ref):
        x = jnp.dot(f_ref[...], ws_ref[...],
                    preferred_element_type=jnp.float32, precision=_HI)
        x = x + bs_ref[...]
        o_ref[...] = jnp.dot(x, w0_ref[...],
                             preferred_element_type=jnp.float32, precision=_HI)

    return pl.pallas_call(
        body,
        grid=(n // bn,),
        in_specs=[
            pl.BlockSpec((bn, d), lambda i: (i, 0)),
            pl.BlockSpec((d, h), lambda i: (0, 0)),
            pl.BlockSpec((1, h), lambda i: (0, 0)),
            pl.BlockSpec((h, mid), lambda i: (0, 0)),
        ],
        out_specs=pl.BlockSpec((bn, mid), lambda i: (i, 0)),
        out_shape=jax.ShapeDtypeStruct((n, mid), jnp.float32),
    )(features, Ws, bs, W0)


def _tc_relu(p0, p1, b0, n):
    mid = p0.shape[1]
    bn = 2048

    def body(a_ref, b_ref, bias_ref, o_ref):
        o_ref[...] = jnp.maximum(a_ref[...] + b_ref[...] + bias_ref[...], 0.0)

    return pl.pallas_call(
        body,
        grid=(n // bn,),
        in_specs=[
            pl.BlockSpec((bn, mid), lambda i: (i, 0)),
            pl.BlockSpec((bn, mid), lambda i: (i, 0)),
            pl.BlockSpec((1, mid), lambda i: (0, 0)),
        ],
        out_specs=pl.BlockSpec((bn, mid), lambda i: (i, 0)),
        out_shape=jax.ShapeDtypeStruct((n, mid), jnp.float32),
    )(p0, p1, b0)


def _tc_out(q0, q1, W1, b1, n):
    mid = q0.shape[1]
    h = W1.shape[1]
    bn = 2000

    def body(a_ref, b_ref, w_ref, bias_ref, o_ref):
        x = a_ref[...] + b_ref[...]
        o_ref[...] = jnp.dot(x, w_ref[...],
                             preferred_element_type=jnp.float32,
                             precision=_HI) + bias_ref[...]

    return pl.pallas_call(
        body,
        grid=(n // bn,),
        in_specs=[
            pl.BlockSpec((bn, mid), lambda i: (i, 0)),
            pl.BlockSpec((bn, mid), lambda i: (i, 0)),
            pl.BlockSpec((mid, h), lambda i: (0, 0)),
            pl.BlockSpec((1, h), lambda i: (0, 0)),
        ],
        out_specs=pl.BlockSpec((bn, h), lambda i: (i, 0)),
        out_shape=jax.ShapeDtypeStruct((n, h), jnp.float32),
    )(q0, q1, W1, b1)


def kernel(features, rows, cols, vals, W_start, b_start, W0, b0, W1, b1):
    n = features.shape[0]
    nnz = rows.shape[0]
    k_per_w = -(-nnz // (NW * CH))
    k_per_w = -(-k_per_w // 8) * 8
    nnz_pad = NW * k_per_w * CH
    n_pad = -(-n // 2048) * 2048
    pad = nnz_pad - nnz
    rows2d = jnp.pad(rows, (0, pad)).reshape(NW * k_per_w, CH)
    cols2d = jnp.pad(cols, (0, pad)).reshape(NW * k_per_w, CH)
    vals1d = jnp.pad(vals, (0, pad))

    zeros_hbm = jnp.zeros((n_pad, W0.shape[1]), jnp.float32)
    f_pad = jnp.pad(features, ((0, n_pad - n), (0, 0)))
    z = _tc_in(f_pad, W_start, b_start.reshape(1, -1), W0)
    p = _spmm_sc(z, rows2d, cols2d, vals1d, zeros_hbm, n_pad)
    h = _tc_relu(p[0], p[1], b0.reshape(1, -1), n_pad)
    q = _spmm_sc(h, rows2d, cols2d, vals1d, zeros_hbm, n_pad)
    return _tc_out(q[0], q[1], W1, b1.reshape(1, -1), n)


# speedup vs baseline: 3.5512x; 1.0795x over previous
import functools

import jax
import jax.numpy as jnp
from jax import lax
from jax.experimental import pallas as pl
from jax.experimental.pallas import tpu as pltpu
from jax.experimental.pallas import tpu_sc as plsc

NC = 2
NS = 16
NW = NC * NS
CH = 128
NB = 4

_GDN = lax.GatherDimensionNumbers(
    offset_dims=(), collapsed_slice_dims=(0,), start_index_map=(0,))


def _spmm_sc(z, rows2d, cols2d, vals2d, zeros_hbm, n_pad):
    F = z.shape[1]
    K = rows2d.shape[0] // NW
    npad = n_pad
    rpt = npad // NS
    mesh = plsc.VectorSubcoreMesh(core_axis_name="c", subcore_axis_name="s")

    @functools.partial(
        pl.kernel,
        mesh=mesh,
        compiler_params=pltpu.CompilerParams(use_tc_tiling_on_sc=False),
        out_type=jax.ShapeDtypeStruct((NC, npad, F), jnp.float32),
        scratch_types=[
            pltpu.VMEM((NB, CH), jnp.int32),
            pltpu.VMEM((NB, CH), jnp.int32),
            pltpu.VMEM((K * CH,), jnp.float32),
            pltpu.VMEM((NB, CH, F), jnp.float32),
            pltpu.VMEM_SHARED((npad, F), jnp.float32),
            pltpu.VMEM_SHARED((npad, F), jnp.float32),
            pltpu.SemaphoreType.DMA((NB,)),
            pltpu.SemaphoreType.DMA((NB,)),
            pltpu.SemaphoreType.DMA((NB,)),
        ],
    )
    def k(z_hbm, rows_hbm, cols_hbm, vals_hbm, zer_hbm, out_hbm,
          colb, rowb, vals_v, gbuf, zs, acc, sem_i, sem_g, sem_s):
        c = lax.axis_index("c")
        s = lax.axis_index("s")
        wid = s * NC + c

        pltpu.sync_copy(vals_hbm.at[pl.ds(wid * K * CH, K * CH)], vals_v)
        pltpu.sync_copy(z_hbm.at[pl.ds(s * rpt, rpt)],
                        zs.at[pl.ds(s * rpt, rpt)])
        pltpu.sync_copy(zer_hbm.at[pl.ds(s * rpt, rpt)],
                        acc.at[pl.ds(s * rpt, rpt)])
        plsc.subcore_barrier()

        def i_issue(j, b):
            pltpu.async_copy(cols_hbm.at[wid * K + j], colb.at[b], sem_i.at[b])
            pltpu.async_copy(rows_hbm.at[wid * K + j], rowb.at[b], sem_i.at[b])

        def i_wait(j, b):
            pltpu.make_async_copy(
                cols_hbm.at[wid * K + j], colb.at[b], sem_i.at[b]).wait()
            pltpu.make_async_copy(
                rows_hbm.at[wid * K + j], rowb.at[b], sem_i.at[b]).wait()

        def g_issue(j, b):
            pltpu.async_copy(zs.at[colb.at[b]], gbuf.at[b], sem_g.at[b])

        def g_wait(j, b):
            pltpu.make_async_copy(
                zs.at[colb.at[b]], gbuf.at[b], sem_g.at[b]).wait()

        def s_issue(j, b):
            pltpu.async_copy(gbuf.at[b], acc.at[rowb.at[b]], sem_s.at[b],
                             add=True)

        def s_wait(j, b):
            pltpu.make_async_copy(
                gbuf.at[b], acc.at[rowb.at[b]], sem_s.at[b]).wait()

        def scale(j, b):
            def edge16(g, icarry):
                vv = vals_v[pl.ds(j * CH + g * 16, 16)]
                for e in range(16):
                    v = lax.gather(
                        vv, jnp.full((16, 1), e, jnp.int32),
                        _GDN, slice_sizes=(1,),
                        mode=lax.GatherScatterMode.PROMISE_IN_BOUNDS)
                    row = g * 16 + e
                    for cc in range(F // 16):
                        gbuf[b, row, pl.ds(cc * 16, 16)] = (
                            gbuf[b, row, pl.ds(cc * 16, 16)] * v)
                return icarry
            lax.fori_loop(0, CH // 16, edge16, 0)

        i_issue(0, 0)
        i_issue(1, 1)
        i_wait(0, 0)
        g_issue(0, 0)
        for j in range(2):
            b = j % NB
            g_wait(j, b)
            scale(j, b)
            s_issue(j, b)
            i_issue(j + 2, (j + 2) % NB)
            i_wait(j + 1, (j + 1) % NB)
            g_issue(j + 1, (j + 1) % NB)

        def main(o, carry):
            for bi in range(NB):
                j = 2 + o * NB + bi
                b = (2 + bi) % NB
                g_wait(j, b)
                scale(j, b)
                s_issue(j, b)
                b2 = (b + 2) % NB
                s_wait(j - 2, b2)
                i_issue(j + 2, b2)
                b1 = (b + 1) % NB
                i_wait(j + 1, b1)
                g_issue(j + 1, b1)
            return carry
        lax.fori_loop(0, (K - 4) // NB, main, 0)

        j = K - 2
        b = j % NB
        g_wait(j, b)
        scale(j, b)
        s_issue(j, b)
        s_wait(j - 2, (j - 2) % NB)
        i_wait(j + 1, (j + 1) % NB)
        g_issue(j + 1, (j + 1) % NB)

        j = K - 1
        b = j % NB
        g_wait(j, b)
        scale(j, b)
        s_issue(j, b)
        s_wait(j - 2, (j - 2) % NB)

        s_wait(K - 2, (K - 2) % NB)
        s_wait(K - 1, (K - 1) % NB)
        plsc.subcore_barrier()

        pltpu.sync_copy(acc.at[pl.ds(s * rpt, rpt)],
                        out_hbm.at[c, pl.ds(s * rpt, rpt)])

    return k(z, rows2d, cols2d, vals2d, zeros_hbm)


def _tc_in(features, Ws, bs, W0, n_pad):
    n, d = features.shape
    h = Ws.shape[1]
    mid = W0.shape[1]
    bn = 2000

    def body(f_ref, ws_ref, bs_ref, w0_ref, o_ref):
        x = jnp.dot(f_ref[...], ws_ref[...],
                    preferred_element_type=jnp.float32)
        x = x + bs_ref[...]
        o_ref[...] = jnp.dot(x, w0_ref[...],
                             preferred_element_type=jnp.float32)

    return pl.pallas_call(
        body,
        grid=(n // bn,),
        in_specs=[
            pl.BlockSpec((bn, d), lambda i: (i, 0)),
            pl.BlockSpec((d, h), lambda i: (0, 0)),
            pl.BlockSpec((1, h), lambda i: (0, 0)),
            pl.BlockSpec((h, mid), lambda i: (0, 0)),
        ],
        out_specs=pl.BlockSpec((bn, mid), lambda i: (i, 0)),
        out_shape=jax.ShapeDtypeStruct((n_pad, mid), jnp.float32),
    )(features, Ws, bs, W0)


def _tc_relu(p, b0, n, n_pad):
    mid = p.shape[2]
    bn = 2000

    def body(a_ref, b_ref, bias_ref, o_ref):
        o_ref[...] = jnp.maximum(a_ref[0] + b_ref[0] + bias_ref[...], 0.0)

    return pl.pallas_call(
        body,
        grid=(n // bn,),
        in_specs=[
            pl.BlockSpec((1, bn, mid), lambda i: (0, i, 0)),
            pl.BlockSpec((1, bn, mid), lambda i: (1, i, 0)),
            pl.BlockSpec((1, mid), lambda i: (0, 0)),
        ],
        out_specs=pl.BlockSpec((bn, mid), lambda i: (i, 0)),
        out_shape=jax.ShapeDtypeStruct((n_pad, mid), jnp.float32),
    )(p, p, b0)


def _tc_out(q, W1, b1, n):
    mid = q.shape[2]
    h = W1.shape[1]
    bn = 2000

    def body(a_ref, b_ref, w_ref, bias_ref, o_ref):
        x = a_ref[0] + b_ref[0]
        o_ref[...] = jnp.dot(x, w_ref[...],
                             preferred_element_type=jnp.float32) + bias_ref[...]

    return pl.pallas_call(
        body,
        grid=(n // bn,),
        in_specs=[
            pl.BlockSpec((1, bn, mid), lambda i: (0, i, 0)),
            pl.BlockSpec((1, bn, mid), lambda i: (1, i, 0)),
            pl.BlockSpec((mid, h), lambda i: (0, 0)),
            pl.BlockSpec((1, h), lambda i: (0, 0)),
        ],
        out_specs=pl.BlockSpec((bn, h), lambda i: (i, 0)),
        out_shape=jax.ShapeDtypeStruct((n, h), jnp.float32),
    )(q, q, W1, b1)


def kernel(features, rows, cols, vals, W_start, b_start, W0, b0, W1, b1):
    n = features.shape[0]
    nnz = rows.shape[0]
    k_per_w = -(-nnz // (NW * CH))
    k_per_w = -(-k_per_w // 8) * 8
    nnz_pad = NW * k_per_w * CH
    n_pad = -(-n // (NS * 8)) * (NS * 8)
    pad = nnz_pad - nnz
    rows2d = jnp.pad(rows, (0, pad)).reshape(NW * k_per_w, CH)
    cols2d = jnp.pad(cols, (0, pad)).reshape(NW * k_per_w, CH)
    vals1d = jnp.pad(vals, (0, pad))

    zeros_hbm = jnp.zeros((n_pad, W0.shape[1]), jnp.float32)
    z = _tc_in(features, W_start, b_start.reshape(1, -1), W0, n_pad)
    p = _spmm_sc(z, rows2d, cols2d, vals1d, zeros_hbm, n_pad)
    h = _tc_relu(p, b0.reshape(1, -1), n, n_pad)
    q = _spmm_sc(h, rows2d, cols2d, vals1d, zeros_hbm, n_pad)
    return _tc_out(q, W1, b1.reshape(1, -1), n)
